# baseline (speedup 1.0000x reference)
import functools

import jax
import jax.numpy as jnp
import numpy as np
from jax import lax
from jax.experimental import pallas as pl
from jax.experimental.pallas import tpu as pltpu
from jax.experimental.pallas import tpu_sc as plsc

N_NODES = 10000
N_PAD = 10240
N_EDGES = 160000
D = 16
NUM_BASIS = 10
TANH_NORM = 1.5927812
RELU_NORM = float(np.sqrt(2.0))

NC, NS = 2, 16
NW = NC * NS
EP = 163840
PER_W = EP // NW
CHUNK = 128
NCHUNK = PER_W // CHUNK
ROWS_PER_TILE = N_PAD // NS

BLK = 2048
GRID = EP // BLK


def _gather_body(src_hbm, dst_hbm, x_hbm, xs_hbm, xd_hbm,
                 idx_v, rows_v, sem):
  c = lax.axis_index("c")
  s = lax.axis_index("s")
  base = (c * NS + s) * PER_W

  def chunk(j, carry):
    off = pl.multiple_of(base + j * CHUNK, CHUNK)
    pltpu.sync_copy(src_hbm.at[pl.ds(off, CHUNK)], idx_v)
    pltpu.async_copy(x_hbm.at[idx_v], rows_v, sem).wait()
    pltpu.sync_copy(rows_v, xs_hbm.at[pl.ds(off, CHUNK)])
    pltpu.sync_copy(dst_hbm.at[pl.ds(off, CHUNK)], idx_v)
    pltpu.async_copy(x_hbm.at[idx_v], rows_v, sem).wait()
    pltpu.sync_copy(rows_v, xd_hbm.at[pl.ds(off, CHUNK)])
    return carry

  lax.fori_loop(0, NCHUNK, chunk, 0)


def _scatter_body(dst_hbm, eo_hbm, out_hbm, idx_v, rows_v, zbuf_v, acc_sh, sem):
  c = lax.axis_index("c")
  s = lax.axis_index("s")
  base = (c * NS + s) * PER_W
  row0 = s * ROWS_PER_TILE

  zero = jnp.zeros((D,), jnp.float32)

  def zi(i, carry):
    zbuf_v[i, :] = zero
    return carry

  lax.fori_loop(0, ROWS_PER_TILE, zi, 0)
  pltpu.sync_copy(zbuf_v, acc_sh.at[pl.ds(row0, ROWS_PER_TILE)])
  plsc.subcore_barrier()

  def chunk(j, carry):
    off = pl.multiple_of(base + j * CHUNK, CHUNK)
    pltpu.sync_copy(dst_hbm.at[pl.ds(off, CHUNK)], idx_v)
    pltpu.sync_copy(eo_hbm.at[pl.ds(off, CHUNK)], rows_v)
    pltpu.sync_copy(rows_v, acc_sh.at[idx_v], add=True)
    return carry

  lax.fori_loop(0, NCHUNK, chunk, 0)
  plsc.subcore_barrier()
  pltpu.sync_copy(acc_sh.at[pl.ds(row0, ROWS_PER_TILE)], zbuf_v)
  pltpu.sync_copy(zbuf_v, out_hbm.at[c, pl.ds(row0, ROWS_PER_TILE)])


def _dense_body(emb_ref, xs_ref, xd_ref, nrm_ref, w1a_ref, w2a_ref,
                w1bs_ref, w1bd_ref, w2b_ref, r_ref, s_ref, out_ref):
  f32 = jnp.float32
  e = emb_ref[...]
  h1 = jax.nn.relu(jnp.dot(e, w1a_ref[...], preferred_element_type=f32, precision=jax.lax.Precision.HIGHEST))
  h2 = jax.nn.relu(jnp.dot(e, w2a_ref[...], preferred_element_type=f32, precision=jax.lax.Precision.HIGHEST))
  m1 = (jnp.dot(xs_ref[...], w1bs_ref[...], preferred_element_type=f32, precision=jax.lax.Precision.HIGHEST)
        + jnp.dot(xd_ref[...], w1bd_ref[...], preferred_element_type=f32, precision=jax.lax.Precision.HIGHEST))
  h1r = jnp.dot(h1, r_ref[...], preferred_element_type=f32, precision=jax.lax.Precision.HIGHEST)
  ef = jnp.dot(h1r * m1, s_ref[...], preferred_element_type=f32, precision=jax.lax.Precision.HIGHEST)
  m2 = jnp.dot(ef, w2b_ref[...], preferred_element_type=f32, precision=jax.lax.Precision.HIGHEST)
  h2r = jnp.dot(h2, r_ref[...], preferred_element_type=f32, precision=jax.lax.Precision.HIGHEST)
  g = jnp.dot(h2r * m2, s_ref[...], preferred_element_type=f32, precision=jax.lax.Precision.HIGHEST)
  out_ref[...] = (TANH_NORM * jnp.tanh(g)) * nrm_ref[...]


def _combine_body(p_ref, o_ref):
  o_ref[...] = p_ref[0, :N_NODES] + p_ref[1, :N_NODES]


def kernel(x, edge_index, edge_vec, emb, norm, num_nodes,
           fc_w1, fc_w2, fc2_w1, fc2_w2):
  del edge_vec, num_nodes
  f32 = jnp.float32
  src = edge_index[0]
  dst = edge_index[1]
  pad = EP - N_EDGES
  srcp = jnp.pad(src, (0, pad))
  dstp = jnp.pad(dst, (0, pad))
  embp = jnp.pad(emb, ((0, pad), (0, 0)))
  nrmp = jnp.pad(norm, (0, pad)).reshape(EP, 1)
  xp = jnp.pad(x, ((0, N_PAD - N_NODES), (0, 0)))

  w1a = fc_w1 * (RELU_NORM / np.sqrt(NUM_BASIS))
  w2a = fc2_w1 * (RELU_NORM / np.sqrt(NUM_BASIS))
  s1 = (1.0 / np.sqrt(fc_w2.shape[0])) * (1.0 / np.sqrt(2 * D))
  w1b = fc_w2.reshape(D, 2 * D, D).transpose(1, 0, 2).reshape(2 * D, D * D) * s1
  w1bs, w1bd = w1b[:D], w1b[D:]
  s2 = (1.0 / np.sqrt(fc2_w2.shape[0])) * (1.0 / np.sqrt(D))
  w2b = fc2_w2.reshape(D, D, D).transpose(1, 0, 2).reshape(D, D * D) * s2
  r_mat = jnp.repeat(jnp.eye(D, dtype=f32), D, axis=1)
  s_mat = jnp.tile(jnp.eye(D, dtype=f32), (D, 1))

  mesh = plsc.VectorSubcoreMesh(core_axis_name="c", subcore_axis_name="s",
                                num_cores=NC, num_subcores=NS)

  gather = pl.kernel(
      _gather_body,
      out_type=(jax.ShapeDtypeStruct((EP, D), f32),
                jax.ShapeDtypeStruct((EP, D), f32)),
      mesh=mesh,
      scratch_types=[pltpu.VMEM((CHUNK,), jnp.int32),
                     pltpu.VMEM((CHUNK, D), f32),
                     pltpu.SemaphoreType.DMA],
      compiler_params=pltpu.CompilerParams(use_tc_tiling_on_sc=False),
  )
  xs, xd = gather(srcp, dstp, xp)

  dense = pl.pallas_call(
      _dense_body,
      grid=(GRID,),
      in_specs=[
          pl.BlockSpec((BLK, NUM_BASIS), lambda i: (i, 0)),
          pl.BlockSpec((BLK, D), lambda i: (i, 0)),
          pl.BlockSpec((BLK, D), lambda i: (i, 0)),
          pl.BlockSpec((BLK, 1), lambda i: (i, 0)),
          pl.BlockSpec((NUM_BASIS, D), lambda i: (0, 0)),
          pl.BlockSpec((NUM_BASIS, D), lambda i: (0, 0)),
          pl.BlockSpec((D, D * D), lambda i: (0, 0)),
          pl.BlockSpec((D, D * D), lambda i: (0, 0)),
          pl.BlockSpec((D, D * D), lambda i: (0, 0)),
          pl.BlockSpec((D, D * D), lambda i: (0, 0)),
          pl.BlockSpec((D * D, D), lambda i: (0, 0)),
      ],
      out_specs=pl.BlockSpec((BLK, D), lambda i: (i, 0)),
      out_shape=jax.ShapeDtypeStruct((EP, D), f32),
      compiler_params=pltpu.CompilerParams(
          dimension_semantics=("parallel",)),
  )
  eo = dense(embp, xs, xd, nrmp, w1a, w2a, w1bs, w1bd, w2b, r_mat, s_mat)

  scatter = pl.kernel(
      _scatter_body,
      out_type=jax.ShapeDtypeStruct((NC, N_PAD, D), f32),
      mesh=mesh,
      scratch_types=[pltpu.VMEM((CHUNK,), jnp.int32),
                     pltpu.VMEM((CHUNK, D), f32),
                     pltpu.VMEM((ROWS_PER_TILE, D), f32),
                     pltpu.VMEM_SHARED((N_PAD, D), f32),
                     pltpu.SemaphoreType.DMA],
      compiler_params=pltpu.CompilerParams(use_tc_tiling_on_sc=False),
  )
  parts = scatter(dstp, eo)

  combine = pl.pallas_call(
      _combine_body,
      out_shape=jax.ShapeDtypeStruct((N_NODES, D), f32),
  )
  return combine(parts)


# speedup vs baseline: 2.6207x; 2.6207x over previous
import functools

import jax
import jax.numpy as jnp
import numpy as np
from jax import lax
from jax.experimental import pallas as pl
from jax.experimental.pallas import tpu as pltpu
from jax.experimental.pallas import tpu_sc as plsc

N_NODES = 10000
N_PAD = 10240
N_EDGES = 160000
D = 16
NUM_BASIS = 10
TANH_NORM = 1.5927812
RELU_NORM = float(np.sqrt(2.0))

NC, NS = 2, 16
NW = NC * NS
EP = 163840
PER_W = EP // NW
CHUNK = 128
NCHUNK = PER_W // CHUNK
ROWS_PER_TILE = N_PAD // NS

BLK = 2048
GRID = EP // BLK


def _gather_body(src_hbm, dst_hbm, x_hbm, xs_hbm, xd_hbm,
                 idx_v, rows_v, sem):
  c = lax.axis_index("c")
  s = lax.axis_index("s")
  base = (c * NS + s) * PER_W

  def chunk(j, carry):
    off = pl.multiple_of(base + j * CHUNK, CHUNK)
    pltpu.sync_copy(src_hbm.at[pl.ds(off, CHUNK)], idx_v)
    pltpu.async_copy(x_hbm.at[idx_v], rows_v, sem).wait()
    pltpu.sync_copy(rows_v, xs_hbm.at[pl.ds(off, CHUNK)])
    pltpu.sync_copy(dst_hbm.at[pl.ds(off, CHUNK)], idx_v)
    pltpu.async_copy(x_hbm.at[idx_v], rows_v, sem).wait()
    pltpu.sync_copy(rows_v, xd_hbm.at[pl.ds(off, CHUNK)])
    return carry

  lax.fori_loop(0, NCHUNK, chunk, 0)


def _scatter_body(dst_hbm, eo_hbm, out_hbm, idx_v, rows_v, zbuf_v, acc_sh, sem):
  c = lax.axis_index("c")
  s = lax.axis_index("s")
  base = (c * NS + s) * PER_W
  row0 = s * ROWS_PER_TILE

  zero = jnp.zeros((D,), jnp.float32)

  def zi(i, carry):
    zbuf_v[i, :] = zero
    return carry

  lax.fori_loop(0, ROWS_PER_TILE, zi, 0)
  pltpu.sync_copy(zbuf_v, acc_sh.at[pl.ds(row0, ROWS_PER_TILE)])
  plsc.subcore_barrier()

  def chunk(j, carry):
    off = pl.multiple_of(base + j * CHUNK, CHUNK)
    pltpu.sync_copy(dst_hbm.at[pl.ds(off, CHUNK)], idx_v)
    pltpu.sync_copy(eo_hbm.at[pl.ds(off, CHUNK)], rows_v)
    pltpu.sync_copy(rows_v, acc_sh.at[idx_v], add=True)
    return carry

  lax.fori_loop(0, NCHUNK, chunk, 0)
  plsc.subcore_barrier()
  pltpu.sync_copy(acc_sh.at[pl.ds(row0, ROWS_PER_TILE)], zbuf_v)
  pltpu.sync_copy(zbuf_v, out_hbm.at[c, pl.ds(row0, ROWS_PER_TILE)])


def _dense_body(emb_ref, xs_ref, xd_ref, nrm_ref, w1a_ref, w2a_ref,
                w1bs_ref, w1bd_ref, w2b_ref, r_ref, s_ref, out_ref):
  f32 = jnp.float32
  e = emb_ref[...]
  h1 = jax.nn.relu(jnp.dot(e, w1a_ref[...], preferred_element_type=f32))
  h2 = jax.nn.relu(jnp.dot(e, w2a_ref[...], preferred_element_type=f32))
  m1 = (jnp.dot(xs_ref[...], w1bs_ref[...], preferred_element_type=f32)
        + jnp.dot(xd_ref[...], w1bd_ref[...], preferred_element_type=f32))
  h1r = jnp.dot(h1, r_ref[...], preferred_element_type=f32)
  ef = jnp.dot(h1r * m1, s_ref[...], preferred_element_type=f32)
  m2 = jnp.dot(ef, w2b_ref[...], preferred_element_type=f32)
  h2r = jnp.dot(h2, r_ref[...], preferred_element_type=f32)
  g = jnp.dot(h2r * m2, s_ref[...], preferred_element_type=f32)
  out_ref[...] = (TANH_NORM * jnp.tanh(g)) * nrm_ref[...]


def _combine_body(p_ref, o_ref):
  o_ref[...] = p_ref[0, :N_NODES] + p_ref[1, :N_NODES]


def kernel(x, edge_index, edge_vec, emb, norm, num_nodes,
           fc_w1, fc_w2, fc2_w1, fc2_w2):
  del edge_vec, num_nodes
  f32 = jnp.float32
  src = edge_index[0]
  dst = edge_index[1]
  pad = EP - N_EDGES
  srcp = jnp.pad(src, (0, pad))
  dstp = jnp.pad(dst, (0, pad))
  embp = jnp.pad(emb, ((0, pad), (0, 0)))
  nrmp = jnp.pad(norm, (0, pad)).reshape(EP, 1)
  xp = jnp.pad(x, ((0, N_PAD - N_NODES), (0, 0)))

  w1a = fc_w1 * (RELU_NORM / np.sqrt(NUM_BASIS))
  w2a = fc2_w1 * (RELU_NORM / np.sqrt(NUM_BASIS))
  s1 = (1.0 / np.sqrt(fc_w2.shape[0])) * (1.0 / np.sqrt(2 * D))
  w1b = fc_w2.reshape(D, 2 * D, D).transpose(1, 0, 2).reshape(2 * D, D * D) * s1
  w1bs, w1bd = w1b[:D], w1b[D:]
  s2 = (1.0 / np.sqrt(fc2_w2.shape[0])) * (1.0 / np.sqrt(D))
  w2b = fc2_w2.reshape(D, D, D).transpose(1, 0, 2).reshape(D, D * D) * s2
  r_mat = jnp.repeat(jnp.eye(D, dtype=f32), D, axis=1)
  s_mat = jnp.tile(jnp.eye(D, dtype=f32), (D, 1))

  mesh = plsc.VectorSubcoreMesh(core_axis_name="c", subcore_axis_name="s",
                                num_cores=NC, num_subcores=NS)

  gather = pl.kernel(
      _gather_body,
      out_type=(jax.ShapeDtypeStruct((EP, D), f32),
                jax.ShapeDtypeStruct((EP, D), f32)),
      mesh=mesh,
      scratch_types=[pltpu.VMEM((CHUNK,), jnp.int32),
                     pltpu.VMEM((CHUNK, D), f32),
                     pltpu.SemaphoreType.DMA],
      compiler_params=pltpu.CompilerParams(use_tc_tiling_on_sc=False),
  )
  xs, xd = gather(srcp, dstp, xp)

  dense = pl.pallas_call(
      _dense_body,
      grid=(GRID,),
      in_specs=[
          pl.BlockSpec((BLK, NUM_BASIS), lambda i: (i, 0)),
          pl.BlockSpec((BLK, D), lambda i: (i, 0)),
          pl.BlockSpec((BLK, D), lambda i: (i, 0)),
          pl.BlockSpec((BLK, 1), lambda i: (i, 0)),
          pl.BlockSpec((NUM_BASIS, D), lambda i: (0, 0)),
          pl.BlockSpec((NUM_BASIS, D), lambda i: (0, 0)),
          pl.BlockSpec((D, D * D), lambda i: (0, 0)),
          pl.BlockSpec((D, D * D), lambda i: (0, 0)),
          pl.BlockSpec((D, D * D), lambda i: (0, 0)),
          pl.BlockSpec((D, D * D), lambda i: (0, 0)),
          pl.BlockSpec((D * D, D), lambda i: (0, 0)),
      ],
      out_specs=pl.BlockSpec((BLK, D), lambda i: (i, 0)),
      out_shape=jax.ShapeDtypeStruct((EP, D), f32),
      compiler_params=pltpu.CompilerParams(
          dimension_semantics=("parallel",)),
  )
  eo = dense(embp, xs, xd, nrmp, w1a, w2a, w1bs, w1bd, w2b, r_mat, s_mat)

  scatter = pl.kernel(
      _scatter_body,
      out_type=jax.ShapeDtypeStruct((NC, N_PAD, D), f32),
      mesh=mesh,
      scratch_types=[pltpu.VMEM((CHUNK,), jnp.int32),
                     pltpu.VMEM((CHUNK, D), f32),
                     pltpu.VMEM((ROWS_PER_TILE, D), f32),
                     pltpu.VMEM_SHARED((N_PAD, D), f32),
                     pltpu.SemaphoreType.DMA],
      compiler_params=pltpu.CompilerParams(use_tc_tiling_on_sc=False),
  )
  parts = scatter(dstp, eo)

  combine = pl.pallas_call(
      _combine_body,
      out_shape=jax.ShapeDtypeStruct((N_NODES, D), f32),
  )
  return combine(parts)


# speedup vs baseline: 3.1012x; 1.1833x over previous
import functools

import jax
import jax.numpy as jnp
import numpy as np
from jax import lax
from jax.experimental import pallas as pl
from jax.experimental.pallas import tpu as pltpu
from jax.experimental.pallas import tpu_sc as plsc

N_NODES = 10000
N_PAD = 10240
N_EDGES = 160000
D = 16
NUM_BASIS = 10
TANH_NORM = 1.5927812
RELU_NORM = float(np.sqrt(2.0))

NC, NS = 2, 16
NW = NC * NS
EP = 163840
PER_W = EP // NW
CHUNK = 128
NCHUNK = PER_W // CHUNK
ROWS_PER_TILE = N_PAD // NS

BLK = 2048
GRID = EP // BLK


def _gather_body(src_hbm, dst_hbm, x_hbm, xs_hbm, xd_hbm,
                 idx_v, rows_v, sem):
  c = lax.axis_index("c")
  s = lax.axis_index("s")
  w = s * NC + c
  rbase = w * NCHUNK
  ebase = w * PER_W

  for ihbm, ohbm in ((src_hbm, xs_hbm), (dst_hbm, xd_hbm)):
    pltpu.sync_copy(ihbm.at[pl.ds(rbase, NCHUNK)], idx_v)

    def fire(j, carry):
      pltpu.async_copy(x_hbm.at[idx_v.at[j]],
                       rows_v.at[pl.ds(j * CHUNK, CHUNK)], sem)
      return carry

    lax.fori_loop(0, NCHUNK, fire, 0)
    pltpu.make_async_copy(ohbm.at[pl.ds(ebase, PER_W)], rows_v, sem).wait()
    pltpu.sync_copy(rows_v, ohbm.at[pl.ds(ebase, PER_W)])


def _scatter_body(dst_hbm, eo_hbm, out_hbm, idx_v, rows_v, zbuf_v, acc_sh,
                  sem_ld, sem_add):
  c = lax.axis_index("c")
  s = lax.axis_index("s")
  w = s * NC + c
  rbase = w * NCHUNK
  ebase = w * PER_W
  row0 = s * ROWS_PER_TILE

  zero = jnp.zeros((D,), jnp.float32)

  def zi(i, carry):
    zbuf_v[i, :] = zero
    return carry

  lax.fori_loop(0, ROWS_PER_TILE, zi, 0)
  pltpu.async_copy(eo_hbm.at[pl.ds(ebase, PER_W)], rows_v, sem_ld)
  pltpu.sync_copy(dst_hbm.at[pl.ds(rbase, NCHUNK)], idx_v)
  pltpu.sync_copy(zbuf_v, acc_sh.at[pl.ds(row0, ROWS_PER_TILE)])
  pltpu.make_async_copy(eo_hbm.at[pl.ds(ebase, PER_W)], rows_v, sem_ld).wait()
  plsc.subcore_barrier()

  def fire(j, carry):
    pltpu.async_copy(rows_v.at[pl.ds(j * CHUNK, CHUNK)],
                     acc_sh.at[idx_v.at[j]], sem_add, add=True)
    return carry

  lax.fori_loop(0, NCHUNK, fire, 0)
  pltpu.make_async_copy(eo_hbm.at[pl.ds(ebase, PER_W)], rows_v, sem_add).wait()
  plsc.subcore_barrier()
  pltpu.sync_copy(acc_sh.at[pl.ds(row0, ROWS_PER_TILE)], zbuf_v)
  pltpu.sync_copy(zbuf_v, out_hbm.at[c, pl.ds(row0, ROWS_PER_TILE)])


def _dense_body(emb_ref, xs_ref, xd_ref, nrm_ref, w1a_ref, w2a_ref,
                w1bs_ref, w1bd_ref, w2b_ref, r_ref, s_ref, out_ref):
  f32 = jnp.float32
  e = emb_ref[...]
  h1 = jax.nn.relu(jnp.dot(e, w1a_ref[...], preferred_element_type=f32))
  h2 = jax.nn.relu(jnp.dot(e, w2a_ref[...], preferred_element_type=f32))
  m1 = (jnp.dot(xs_ref[...], w1bs_ref[...], preferred_element_type=f32)
        + jnp.dot(xd_ref[...], w1bd_ref[...], preferred_element_type=f32))
  h1r = jnp.dot(h1, r_ref[...], preferred_element_type=f32)
  ef = jnp.dot(h1r * m1, s_ref[...], preferred_element_type=f32)
  m2 = jnp.dot(ef, w2b_ref[...], preferred_element_type=f32)
  h2r = jnp.dot(h2, r_ref[...], preferred_element_type=f32)
  g = jnp.dot(h2r * m2, s_ref[...], preferred_element_type=f32)
  out_ref[...] = (TANH_NORM * jnp.tanh(g)) * nrm_ref[...]


def _combine_body(p_ref, o_ref):
  o_ref[...] = p_ref[0, :N_NODES] + p_ref[1, :N_NODES]


def kernel(x, edge_index, edge_vec, emb, norm, num_nodes,
           fc_w1, fc_w2, fc2_w1, fc2_w2):
  del edge_vec, num_nodes
  f32 = jnp.float32
  src = edge_index[0]
  dst = edge_index[1]
  pad = EP - N_EDGES
  srcp = jnp.pad(src, (0, pad))
  dstp = jnp.pad(dst, (0, pad))
  embp = jnp.pad(emb, ((0, pad), (0, 0)))
  nrmp = jnp.pad(norm, (0, pad)).reshape(EP, 1)
  xp = jnp.pad(x, ((0, N_PAD - N_NODES), (0, 0)))

  w1a = fc_w1 * (RELU_NORM / np.sqrt(NUM_BASIS))
  w2a = fc2_w1 * (RELU_NORM / np.sqrt(NUM_BASIS))
  s1 = (1.0 / np.sqrt(fc_w2.shape[0])) * (1.0 / np.sqrt(2 * D))
  w1b = fc_w2.reshape(D, 2 * D, D).transpose(1, 0, 2).reshape(2 * D, D * D) * s1
  w1bs, w1bd = w1b[:D], w1b[D:]
  s2 = (1.0 / np.sqrt(fc2_w2.shape[0])) * (1.0 / np.sqrt(D))
  w2b = fc2_w2.reshape(D, D, D).transpose(1, 0, 2).reshape(D, D * D) * s2
  r_mat = jnp.repeat(jnp.eye(D, dtype=f32), D, axis=1)
  s_mat = jnp.tile(jnp.eye(D, dtype=f32), (D, 1))

  mesh = plsc.VectorSubcoreMesh(core_axis_name="c", subcore_axis_name="s",
                                num_cores=NC, num_subcores=NS)

  gather = pl.kernel(
      _gather_body,
      out_type=(jax.ShapeDtypeStruct((EP, D), f32),
                jax.ShapeDtypeStruct((EP, D), f32)),
      mesh=mesh,
      scratch_types=[pltpu.VMEM((NCHUNK, CHUNK), jnp.int32),
                     pltpu.VMEM((PER_W, D), f32),
                     pltpu.SemaphoreType.DMA],
      compiler_params=pltpu.CompilerParams(use_tc_tiling_on_sc=False),
  )
  xs, xd = gather(srcp.reshape(EP // CHUNK, CHUNK),
                  dstp.reshape(EP // CHUNK, CHUNK), xp)

  dense = pl.pallas_call(
      _dense_body,
      grid=(GRID,),
      in_specs=[
          pl.BlockSpec((BLK, NUM_BASIS), lambda i: (i, 0)),
          pl.BlockSpec((BLK, D), lambda i: (i, 0)),
          pl.BlockSpec((BLK, D), lambda i: (i, 0)),
          pl.BlockSpec((BLK, 1), lambda i: (i, 0)),
          pl.BlockSpec((NUM_BASIS, D), lambda i: (0, 0)),
          pl.BlockSpec((NUM_BASIS, D), lambda i: (0, 0)),
          pl.BlockSpec((D, D * D), lambda i: (0, 0)),
          pl.BlockSpec((D, D * D), lambda i: (0, 0)),
          pl.BlockSpec((D, D * D), lambda i: (0, 0)),
          pl.BlockSpec((D, D * D), lambda i: (0, 0)),
          pl.BlockSpec((D * D, D), lambda i: (0, 0)),
      ],
      out_specs=pl.BlockSpec((BLK, D), lambda i: (i, 0)),
      out_shape=jax.ShapeDtypeStruct((EP, D), f32),
      compiler_params=pltpu.CompilerParams(
          dimension_semantics=("parallel",)),
  )
  eo = dense(embp, xs, xd, nrmp, w1a, w2a, w1bs, w1bd, w2b, r_mat, s_mat)

  scatter = pl.kernel(
      _scatter_body,
      out_type=jax.ShapeDtypeStruct((NC, N_PAD, D), f32),
      mesh=mesh,
      scratch_types=[pltpu.VMEM((NCHUNK, CHUNK), jnp.int32),
                     pltpu.VMEM((PER_W, D), f32),
                     pltpu.VMEM((ROWS_PER_TILE, D), f32),
                     pltpu.VMEM_SHARED((N_PAD, D), f32),
                     pltpu.SemaphoreType.DMA,
                     pltpu.SemaphoreType.DMA],
      compiler_params=pltpu.CompilerParams(use_tc_tiling_on_sc=False),
  )
  parts = scatter(dstp.reshape(EP // CHUNK, CHUNK), eo)

  combine = pl.pallas_call(
      _combine_body,
      out_shape=jax.ShapeDtypeStruct((N_NODES, D), f32),
  )
  return combine(parts)


# speedup vs baseline: 3.6182x; 1.1667x over previous
import functools

import jax
import jax.numpy as jnp
import numpy as np
from jax import lax
from jax.experimental import pallas as pl
from jax.experimental.pallas import tpu as pltpu
from jax.experimental.pallas import tpu_sc as plsc

N_NODES = 10000
N_PAD = 10240
N_EDGES = 160000
D = 16
NUM_BASIS = 10
TANH_NORM = 1.5927812
RELU_NORM = float(np.sqrt(2.0))

NC, NS = 2, 16
NW = NC * NS
EP = 163840
PER_W = EP // NW
CHUNK = 128
NCHUNK = PER_W // CHUNK
ROWS_PER_TILE = N_PAD // NS

BLK = 2048
GRID = EP // BLK


def _gather_body(src_hbm, dst_hbm, x_hbm, xs_hbm, idx_v, rows_v, sem):
  c = lax.axis_index("c")
  s = lax.axis_index("s")
  w = s * NC + c
  rbase = w * NCHUNK
  ebase = w * PER_W

  for ihbm, col0 in ((src_hbm, 0), (dst_hbm, D)):
    pltpu.sync_copy(ihbm.at[pl.ds(rbase, NCHUNK)], idx_v)

    def fire(j, carry):
      pltpu.async_copy(x_hbm.at[idx_v.at[j]],
                       rows_v.at[pl.ds(j * CHUNK, CHUNK)], sem)
      return carry

    lax.fori_loop(0, NCHUNK, fire, 0)
    pltpu.make_async_copy(xs_hbm.at[pl.ds(ebase, PER_W), pl.ds(col0, D)],
                          rows_v, sem).wait()
    pltpu.sync_copy(rows_v, xs_hbm.at[pl.ds(ebase, PER_W), pl.ds(col0, D)])


def _scatter_body(dst_hbm, eo_hbm, out_hbm, idx_v, rows_v, zbuf_v, acc_sh,
                  sem_ld, sem_add):
  c = lax.axis_index("c")
  s = lax.axis_index("s")
  w = s * NC + c
  rbase = w * NCHUNK
  ebase = w * PER_W
  row0 = s * ROWS_PER_TILE

  zero = jnp.zeros((D,), jnp.float32)

  def zi(i, carry):
    zbuf_v[i, :] = zero
    return carry

  lax.fori_loop(0, ROWS_PER_TILE, zi, 0)
  pltpu.async_copy(eo_hbm.at[pl.ds(ebase, PER_W)], rows_v, sem_ld)
  pltpu.sync_copy(dst_hbm.at[pl.ds(rbase, NCHUNK)], idx_v)
  pltpu.sync_copy(zbuf_v, acc_sh.at[pl.ds(row0, ROWS_PER_TILE)])
  pltpu.make_async_copy(eo_hbm.at[pl.ds(ebase, PER_W)], rows_v, sem_ld).wait()
  plsc.subcore_barrier()

  def fire(j, carry):
    pltpu.async_copy(rows_v.at[pl.ds(j * CHUNK, CHUNK)],
                     acc_sh.at[idx_v.at[j]], sem_add, add=True)
    return carry

  lax.fori_loop(0, NCHUNK, fire, 0)
  pltpu.make_async_copy(eo_hbm.at[pl.ds(ebase, PER_W)], rows_v, sem_add).wait()
  plsc.subcore_barrier()
  pltpu.sync_copy(acc_sh.at[pl.ds(row0, ROWS_PER_TILE)], zbuf_v)
  pltpu.sync_copy(zbuf_v, out_hbm.at[c, pl.ds(row0, ROWS_PER_TILE)])


def _dense_body(emb_ref, feat_ref, nrm_ref, whh_ref, w1b_ref, sw2_ref,
                r_ref, s_ref, out_ref):
  f32 = jnp.float32
  hh = jax.nn.relu(jnp.dot(emb_ref[...], whh_ref[...],
                           preferred_element_type=f32))
  h1 = hh[:, :D]
  h2 = hh[:, D:]
  m1 = jnp.dot(feat_ref[...], w1b_ref[...], preferred_element_type=f32)
  h1r = jnp.dot(h1, r_ref[...], preferred_element_type=f32)
  m2 = jnp.dot(h1r * m1, sw2_ref[...], preferred_element_type=f32)
  h2r = jnp.dot(h2, r_ref[...], preferred_element_type=f32)
  g = jnp.dot(h2r * m2, s_ref[...], preferred_element_type=f32)
  out_ref[...] = (TANH_NORM * jnp.tanh(g)) * nrm_ref[...]


def _combine_body(p_ref, o_ref):
  o_ref[...] = p_ref[0, :N_NODES] + p_ref[1, :N_NODES]


def kernel(x, edge_index, edge_vec, emb, norm, num_nodes,
           fc_w1, fc_w2, fc2_w1, fc2_w2):
  del edge_vec, num_nodes
  f32 = jnp.float32
  src = edge_index[0]
  dst = edge_index[1]
  pad = EP - N_EDGES
  srcp = jnp.pad(src, (0, pad))
  dstp = jnp.pad(dst, (0, pad))
  embp = jnp.pad(emb, ((0, pad), (0, 0)))
  nrmp = jnp.pad(norm, (0, pad)).reshape(EP, 1)
  xp = jnp.pad(x, ((0, N_PAD - N_NODES), (0, 0)))

  w1a = fc_w1 * (RELU_NORM / np.sqrt(NUM_BASIS))
  w2a = fc2_w1 * (RELU_NORM / np.sqrt(NUM_BASIS))
  whh = jnp.concatenate([w1a, w2a], axis=1)
  s1 = (1.0 / np.sqrt(fc_w2.shape[0])) * (1.0 / np.sqrt(2 * D))
  w1b = fc_w2.reshape(D, 2 * D, D).transpose(1, 0, 2).reshape(2 * D, D * D) * s1
  s2 = (1.0 / np.sqrt(fc2_w2.shape[0])) * (1.0 / np.sqrt(D))
  w2b = fc2_w2.reshape(D, D, D).transpose(1, 0, 2).reshape(D, D * D) * s2
  r_mat = jnp.repeat(jnp.eye(D, dtype=f32), D, axis=1)
  s_mat = jnp.tile(jnp.eye(D, dtype=f32), (D, 1))
  sw2 = jnp.dot(s_mat, w2b)

  mesh = plsc.VectorSubcoreMesh(core_axis_name="c", subcore_axis_name="s",
                                num_cores=NC, num_subcores=NS)

  gather = pl.kernel(
      _gather_body,
      out_type=jax.ShapeDtypeStruct((EP, 2 * D), f32),
      mesh=mesh,
      scratch_types=[pltpu.VMEM((NCHUNK, CHUNK), jnp.int32),
                     pltpu.VMEM((PER_W, D), f32),
                     pltpu.SemaphoreType.DMA],
      compiler_params=pltpu.CompilerParams(use_tc_tiling_on_sc=False),
  )
  feat = gather(srcp.reshape(EP // CHUNK, CHUNK),
                dstp.reshape(EP // CHUNK, CHUNK), xp)

  dense = pl.pallas_call(
      _dense_body,
      grid=(GRID,),
      in_specs=[
          pl.BlockSpec((BLK, NUM_BASIS), lambda i: (i, 0)),
          pl.BlockSpec((BLK, 2 * D), lambda i: (i, 0)),
          pl.BlockSpec((BLK, 1), lambda i: (i, 0)),
          pl.BlockSpec((NUM_BASIS, 2 * D), lambda i: (0, 0)),
          pl.BlockSpec((2 * D, D * D), lambda i: (0, 0)),
          pl.BlockSpec((D * D, D * D), lambda i: (0, 0)),
          pl.BlockSpec((D, D * D), lambda i: (0, 0)),
          pl.BlockSpec((D * D, D), lambda i: (0, 0)),
      ],
      out_specs=pl.BlockSpec((BLK, D), lambda i: (i, 0)),
      out_shape=jax.ShapeDtypeStruct((EP, D), f32),
      compiler_params=pltpu.CompilerParams(
          dimension_semantics=("parallel",)),
  )
  eo = dense(embp, feat, nrmp, whh, w1b, sw2, r_mat, s_mat)

  scatter = pl.kernel(
      _scatter_body,
      out_type=jax.ShapeDtypeStruct((NC, N_PAD, D), f32),
      mesh=mesh,
      scratch_types=[pltpu.VMEM((NCHUNK, CHUNK), jnp.int32),
                     pltpu.VMEM((PER_W, D), f32),
                     pltpu.VMEM((ROWS_PER_TILE, D), f32),
                     pltpu.VMEM_SHARED((N_PAD, D), f32),
                     pltpu.SemaphoreType.DMA,
                     pltpu.SemaphoreType.DMA],
      compiler_params=pltpu.CompilerParams(use_tc_tiling_on_sc=False),
  )
  parts = scatter(dstp.reshape(EP // CHUNK, CHUNK), eo)

  combine = pl.pallas_call(
      _combine_body,
      out_shape=jax.ShapeDtypeStruct((N_NODES, D), f32),
  )
  return combine(parts)


# speedup vs baseline: 3.8310x; 1.0588x over previous
import jax
import jax.numpy as jnp
import numpy as np
from jax import lax
from jax.experimental import pallas as pl
from jax.experimental.pallas import tpu as pltpu
from jax.experimental.pallas import tpu_sc as plsc

N_NODES = 10000
N_PAD = 10240
N_EDGES = 160000
D = 16
NUM_BASIS = 10
TANH_NORM = 1.5927812
RELU_NORM = float(np.sqrt(2.0))

NC, NS = 2, 16
CHUNK = 128
EDGE_ROWS = N_EDGES // CHUNK
NACT = 25
NCHUNK = EDGE_ROWS // NACT
PER_W = NCHUNK * CHUNK
ROWS_PER_TILE = N_PAD // NS

BLK = 2048
GRID = (N_EDGES + BLK - 1) // BLK


def _gather_body(src_hbm, dst_hbm, x_hbm, feat_hbm, idx_v, rows_v, sem):
  c = lax.axis_index("c")
  s = lax.axis_index("s")
  w = s * NC + c

  @pl.when(w < NACT)
  def _():
    rbase = w * NCHUNK
    ebase = w * PER_W
    for ihbm, col0 in ((src_hbm, 0), (dst_hbm, D)):
      pltpu.sync_copy(ihbm.at[pl.ds(rbase, NCHUNK)], idx_v)

      def fire(j, carry):
        pltpu.async_copy(x_hbm.at[idx_v.at[j]],
                         rows_v.at[pl.ds(j * CHUNK, CHUNK)], sem)
        return carry

      lax.fori_loop(0, NCHUNK, fire, 0)
      pltpu.make_async_copy(feat_hbm.at[pl.ds(ebase, PER_W), pl.ds(col0, D)],
                            rows_v, sem).wait()
      pltpu.sync_copy(rows_v, feat_hbm.at[pl.ds(ebase, PER_W), pl.ds(col0, D)])


def _scatter_body(dst_hbm, eo_hbm, out_hbm, idx_v, rows_v, zbuf_v, acc_sh,
                  sem_ld, sem_add):
  c = lax.axis_index("c")
  s = lax.axis_index("s")
  w = s * NC + c
  row0 = s * ROWS_PER_TILE
  active = w < NACT

  zero = jnp.zeros((D,), jnp.float32)

  def zi(i, carry):
    zbuf_v[i, :] = zero
    return carry

  lax.fori_loop(0, ROWS_PER_TILE, zi, 0)

  @pl.when(active)
  def _():
    pltpu.async_copy(eo_hbm.at[pl.ds(w * PER_W, PER_W)], rows_v, sem_ld)
    pltpu.sync_copy(dst_hbm.at[pl.ds(w * NCHUNK, NCHUNK)], idx_v)

  pltpu.sync_copy(zbuf_v, acc_sh.at[pl.ds(row0, ROWS_PER_TILE)])

  @pl.when(active)
  def _():
    pltpu.make_async_copy(eo_hbm.at[pl.ds(w * PER_W, PER_W)], rows_v,
                          sem_ld).wait()

  plsc.subcore_barrier()

  @pl.when(active)
  def _():
    def fire(j, carry):
      pltpu.async_copy(rows_v.at[pl.ds(j * CHUNK, CHUNK)],
                       acc_sh.at[idx_v.at[j]], sem_add, add=True)
      return carry

    lax.fori_loop(0, NCHUNK, fire, 0)
    pltpu.make_async_copy(eo_hbm.at[pl.ds(w * PER_W, PER_W)], rows_v,
                          sem_add).wait()

  plsc.subcore_barrier()
  pltpu.sync_copy(acc_sh.at[pl.ds(row0, ROWS_PER_TILE)], zbuf_v)
  pltpu.sync_copy(zbuf_v, out_hbm.at[c, pl.ds(row0, ROWS_PER_TILE)])


def _dense_body(emb_ref, feat_ref, nrm_ref, whh_ref, w1b_ref, sw2_ref,
                r_ref, s_ref, out_ref):
  f32 = jnp.float32
  hh = jax.nn.relu(jnp.dot(emb_ref[...], whh_ref[...],
                           preferred_element_type=f32))
  h1 = hh[:, :D]
  h2 = hh[:, D:]
  m1 = jnp.dot(feat_ref[...], w1b_ref[...], preferred_element_type=f32)
  h1r = jnp.dot(h1, r_ref[...], preferred_element_type=f32)
  m2 = jnp.dot(h1r * m1, sw2_ref[...], preferred_element_type=f32)
  h2r = jnp.dot(h2, r_ref[...], preferred_element_type=f32)
  g = jnp.dot(h2r * m2, s_ref[...], preferred_element_type=f32)
  out_ref[...] = (TANH_NORM * jnp.tanh(g)) * nrm_ref[...]


def _combine_body(p_ref, o_ref):
  o_ref[...] = p_ref[0, :N_NODES] + p_ref[1, :N_NODES]


def kernel(x, edge_index, edge_vec, emb, norm, num_nodes,
           fc_w1, fc_w2, fc2_w1, fc2_w2):
  del edge_vec, num_nodes
  f32 = jnp.float32
  src2 = edge_index[0].reshape(EDGE_ROWS, CHUNK)
  dst2 = edge_index[1].reshape(EDGE_ROWS, CHUNK)
  nrm2 = norm.reshape(N_EDGES, 1)

  w1a = fc_w1 * (RELU_NORM / np.sqrt(NUM_BASIS))
  w2a = fc2_w1 * (RELU_NORM / np.sqrt(NUM_BASIS))
  whh = jnp.concatenate([w1a, w2a], axis=1)
  s1 = (1.0 / np.sqrt(fc_w2.shape[0])) * (1.0 / np.sqrt(2 * D))
  w1b = fc_w2.reshape(D, 2 * D, D).transpose(1, 0, 2).reshape(2 * D, D * D) * s1
  s2 = (1.0 / np.sqrt(fc2_w2.shape[0])) * (1.0 / np.sqrt(D))
  w2b = fc2_w2.reshape(D, D, D).transpose(1, 0, 2).reshape(D, D * D) * s2
  r_mat = jnp.repeat(jnp.eye(D, dtype=f32), D, axis=1)
  s_mat = jnp.tile(jnp.eye(D, dtype=f32), (D, 1))
  sw2 = jnp.dot(s_mat, w2b)

  mesh = plsc.VectorSubcoreMesh(core_axis_name="c", subcore_axis_name="s",
                                num_cores=NC, num_subcores=NS)

  gather = pl.kernel(
      _gather_body,
      out_type=jax.ShapeDtypeStruct((N_EDGES, 2 * D), f32),
      mesh=mesh,
      scratch_types=[pltpu.VMEM((NCHUNK, CHUNK), jnp.int32),
                     pltpu.VMEM((PER_W, D), f32),
                     pltpu.SemaphoreType.DMA],
      compiler_params=pltpu.CompilerParams(use_tc_tiling_on_sc=False),
  )
  feat = gather(src2, dst2, x)

  dense = pl.pallas_call(
      _dense_body,
      grid=(GRID,),
      in_specs=[
          pl.BlockSpec((BLK, NUM_BASIS), lambda i: (i, 0)),
          pl.BlockSpec((BLK, 2 * D), lambda i: (i, 0)),
          pl.BlockSpec((BLK, 1), lambda i: (i, 0)),
          pl.BlockSpec((NUM_BASIS, 2 * D), lambda i: (0, 0)),
          pl.BlockSpec((2 * D, D * D), lambda i: (0, 0)),
          pl.BlockSpec((D * D, D * D), lambda i: (0, 0)),
          pl.BlockSpec((D, D * D), lambda i: (0, 0)),
          pl.BlockSpec((D * D, D), lambda i: (0, 0)),
      ],
      out_specs=pl.BlockSpec((BLK, D), lambda i: (i, 0)),
      out_shape=jax.ShapeDtypeStruct((N_EDGES, D), f32),
      compiler_params=pltpu.CompilerParams(
          dimension_semantics=("parallel",)),
  )
  eo = dense(emb, feat, nrm2, whh, w1b, sw2, r_mat, s_mat)

  scatter = pl.kernel(
      _scatter_body,
      out_type=jax.ShapeDtypeStruct((NC, N_PAD, D), f32),
      mesh=mesh,
      scratch_types=[pltpu.VMEM((NCHUNK, CHUNK), jnp.int32),
                     pltpu.VMEM((PER_W, D), f32),
                     pltpu.VMEM((ROWS_PER_TILE, D), f32),
                     pltpu.VMEM_SHARED((N_PAD, D), f32),
                     pltpu.SemaphoreType.DMA,
                     pltpu.SemaphoreType.DMA],
      compiler_params=pltpu.CompilerParams(use_tc_tiling_on_sc=False),
  )
  parts = scatter(dst2, eo)

  combine = pl.pallas_call(
      _combine_body,
      out_shape=jax.ShapeDtypeStruct((N_NODES, D), f32),
  )
  return combine(parts)


# speedup vs baseline: 4.0560x; 1.0588x over previous
import jax
import jax.numpy as jnp
import numpy as np
from jax import lax
from jax.experimental import pallas as pl
from jax.experimental.pallas import tpu as pltpu
from jax.experimental.pallas import tpu_sc as plsc

N_NODES = 10000
N_PAD = 10240
N_EDGES = 160000
D = 16
NUM_BASIS = 10
TANH_NORM = 1.5927812
RELU_NORM = float(np.sqrt(2.0))

NC, NS = 2, 16
CHUNK = 128
EDGE_ROWS = N_EDGES // CHUNK
NACT = 25
NCHUNK = EDGE_ROWS // NACT
PER_W = NCHUNK * CHUNK
ROWS_PER_TILE = N_PAD // NS

BLK = 4096
GRID = (N_EDGES + BLK - 1) // BLK


def _gather_body(src_hbm, dst_hbm, x_hbm, feat_hbm, idx_v, rows_v, sem):
  c = lax.axis_index("c")
  s = lax.axis_index("s")
  w = s * NC + c

  @pl.when(w < NACT)
  def _():
    rbase = w * NCHUNK
    ebase = w * PER_W
    for ihbm, col0 in ((src_hbm, 0), (dst_hbm, D)):
      pltpu.sync_copy(ihbm.at[pl.ds(rbase, NCHUNK)], idx_v)

      def fire(j, carry):
        pltpu.async_copy(x_hbm.at[idx_v.at[j]],
                         rows_v.at[pl.ds(j * CHUNK, CHUNK)], sem)
        return carry

      lax.fori_loop(0, NCHUNK, fire, 0)
      pltpu.make_async_copy(feat_hbm.at[pl.ds(ebase, PER_W), pl.ds(col0, D)],
                            rows_v, sem).wait()
      pltpu.sync_copy(rows_v, feat_hbm.at[pl.ds(ebase, PER_W), pl.ds(col0, D)])


def _scatter_body(dst_hbm, eo_hbm, out_hbm, idx_v, rows_v, zbuf_v, acc_sh,
                  sem_ld, sem_add):
  c = lax.axis_index("c")
  s = lax.axis_index("s")
  w = s * NC + c
  row0 = s * ROWS_PER_TILE
  active = w < NACT

  zero = jnp.zeros((D,), jnp.float32)

  def zi(i, carry):
    zbuf_v[i, :] = zero
    return carry

  lax.fori_loop(0, ROWS_PER_TILE, zi, 0)

  @pl.when(active)
  def _():
    pltpu.async_copy(eo_hbm.at[pl.ds(w * PER_W, PER_W)], rows_v, sem_ld)
    pltpu.sync_copy(dst_hbm.at[pl.ds(w * NCHUNK, NCHUNK)], idx_v)

  pltpu.sync_copy(zbuf_v, acc_sh.at[pl.ds(row0, ROWS_PER_TILE)])

  @pl.when(active)
  def _():
    pltpu.make_async_copy(eo_hbm.at[pl.ds(w * PER_W, PER_W)], rows_v,
                          sem_ld).wait()

  plsc.subcore_barrier()

  @pl.when(active)
  def _():
    def fire(j, carry):
      pltpu.async_copy(rows_v.at[pl.ds(j * CHUNK, CHUNK)],
                       acc_sh.at[idx_v.at[j]], sem_add, add=True)
      return carry

    lax.fori_loop(0, NCHUNK, fire, 0)
    pltpu.make_async_copy(eo_hbm.at[pl.ds(w * PER_W, PER_W)], rows_v,
                          sem_add).wait()

  plsc.subcore_barrier()
  pltpu.sync_copy(acc_sh.at[pl.ds(row0, ROWS_PER_TILE)], zbuf_v)
  pltpu.sync_copy(zbuf_v, out_hbm.at[c, pl.ds(row0, ROWS_PER_TILE)])


def _dense_body(emb_ref, feat_ref, nrm_ref, whh_ref, w1b_ref, sw2_ref,
                r_ref, s_ref, out_ref):
  f32 = jnp.float32
  hh = jax.nn.relu(jnp.dot(emb_ref[...], whh_ref[...],
                           preferred_element_type=f32))
  h1 = hh[:, :D]
  h2 = hh[:, D:]
  m1 = jnp.dot(feat_ref[...], w1b_ref[...], preferred_element_type=f32)
  h1r = jnp.dot(h1, r_ref[...], preferred_element_type=f32)
  m2 = jnp.dot(h1r * m1, sw2_ref[...], preferred_element_type=f32)
  h2r = jnp.dot(h2, r_ref[...], preferred_element_type=f32)
  g = jnp.dot(h2r * m2, s_ref[...], preferred_element_type=f32)
  out_ref[...] = (TANH_NORM * jnp.tanh(g)) * nrm_ref[...]


def _combine_body(p_ref, o_ref):
  o_ref[...] = p_ref[0, :N_NODES] + p_ref[1, :N_NODES]


def kernel(x, edge_index, edge_vec, emb, norm, num_nodes,
           fc_w1, fc_w2, fc2_w1, fc2_w2):
  del edge_vec, num_nodes
  f32 = jnp.float32
  src2 = edge_index[0].reshape(EDGE_ROWS, CHUNK)
  dst2 = edge_index[1].reshape(EDGE_ROWS, CHUNK)
  nrm2 = norm.reshape(N_EDGES, 1)

  w1a = fc_w1 * (RELU_NORM / np.sqrt(NUM_BASIS))
  w2a = fc2_w1 * (RELU_NORM / np.sqrt(NUM_BASIS))
  whh = jnp.concatenate([w1a, w2a], axis=1)
  s1 = (1.0 / np.sqrt(fc_w2.shape[0])) * (1.0 / np.sqrt(2 * D))
  w1b = fc_w2.reshape(D, 2 * D, D).transpose(1, 0, 2).reshape(2 * D, D * D) * s1
  s2 = (1.0 / np.sqrt(fc2_w2.shape[0])) * (1.0 / np.sqrt(D))
  w2b = fc2_w2.reshape(D, D, D).transpose(1, 0, 2).reshape(D, D * D) * s2
  r_mat = jnp.repeat(jnp.eye(D, dtype=f32), D, axis=1)
  s_mat = jnp.tile(jnp.eye(D, dtype=f32), (D, 1))
  sw2 = jnp.dot(s_mat, w2b)

  mesh = plsc.VectorSubcoreMesh(core_axis_name="c", subcore_axis_name="s",
                                num_cores=NC, num_subcores=NS)

  gather = pl.kernel(
      _gather_body,
      out_type=jax.ShapeDtypeStruct((N_EDGES, 2 * D), f32),
      mesh=mesh,
      scratch_types=[pltpu.VMEM((NCHUNK, CHUNK), jnp.int32),
                     pltpu.VMEM((PER_W, D), f32),
                     pltpu.SemaphoreType.DMA],
      compiler_params=pltpu.CompilerParams(use_tc_tiling_on_sc=False),
  )
  feat = gather(src2, dst2, x)

  dense = pl.pallas_call(
      _dense_body,
      grid=(GRID,),
      in_specs=[
          pl.BlockSpec((BLK, NUM_BASIS), lambda i: (i, 0)),
          pl.BlockSpec((BLK, 2 * D), lambda i: (i, 0)),
          pl.BlockSpec((BLK, 1), lambda i: (i, 0)),
          pl.BlockSpec((NUM_BASIS, 2 * D), lambda i: (0, 0)),
          pl.BlockSpec((2 * D, D * D), lambda i: (0, 0)),
          pl.BlockSpec((D * D, D * D), lambda i: (0, 0)),
          pl.BlockSpec((D, D * D), lambda i: (0, 0)),
          pl.BlockSpec((D * D, D), lambda i: (0, 0)),
      ],
      out_specs=pl.BlockSpec((BLK, D), lambda i: (i, 0)),
      out_shape=jax.ShapeDtypeStruct((N_EDGES, D), f32),
      compiler_params=pltpu.CompilerParams(
          dimension_semantics=("parallel",)),
  )
  eo = dense(emb, feat, nrm2, whh, w1b, sw2, r_mat, s_mat)

  scatter = pl.kernel(
      _scatter_body,
      out_type=jax.ShapeDtypeStruct((NC, N_PAD, D), f32),
      mesh=mesh,
      scratch_types=[pltpu.VMEM((NCHUNK, CHUNK), jnp.int32),
                     pltpu.VMEM((PER_W, D), f32),
                     pltpu.VMEM((ROWS_PER_TILE, D), f32),
                     pltpu.VMEM_SHARED((N_PAD, D), f32),
                     pltpu.SemaphoreType.DMA,
                     pltpu.SemaphoreType.DMA],
      compiler_params=pltpu.CompilerParams(use_tc_tiling_on_sc=False),
  )
  parts = scatter(dst2, eo)

  combine = pl.pallas_call(
      _combine_body,
      out_shape=jax.ShapeDtypeStruct((N_NODES, D), f32),
  )
  return combine(parts)


# speedup vs baseline: 4.0601x; 1.0010x over previous
import jax
import jax.numpy as jnp
import numpy as np
from jax import lax
from jax.experimental import pallas as pl
from jax.experimental.pallas import tpu as pltpu
from jax.experimental.pallas import tpu_sc as plsc

N_NODES = 10000
N_PAD = 10240
N_EDGES = 160000
D = 16
NUM_BASIS = 10
TANH_NORM = 1.5927812
RELU_NORM = float(np.sqrt(2.0))

NC, NS = 2, 16
CHUNK = 128
EDGE_ROWS = N_EDGES // CHUNK
NACT = 25
NCHUNK = EDGE_ROWS // NACT
PER_W = NCHUNK * CHUNK
ROWS_PER_TILE = N_PAD // NS

BLK = 4096
GRID = (N_EDGES + BLK - 1) // BLK


def _gather_body(src_hbm, dst_hbm, x_hbm, feat_hbm, idx_v, rows_v, sem):
  c = lax.axis_index("c")
  s = lax.axis_index("s")
  w = s * NC + c

  @pl.when(w < NACT)
  def _():
    ebase = w * PER_W
    for ihbm, col0 in ((src_hbm, 0), (dst_hbm, D)):
      pltpu.sync_copy(ihbm.at[pl.ds(ebase, PER_W)], idx_v)

      def fire(j, carry):
        pltpu.async_copy(x_hbm.at[idx_v.at[pl.ds(j * CHUNK, CHUNK)]],
                         rows_v.at[pl.ds(j * CHUNK, CHUNK)], sem)
        return carry

      lax.fori_loop(0, NCHUNK, fire, 0)
      pltpu.make_async_copy(feat_hbm.at[pl.ds(ebase, PER_W), pl.ds(col0, D)],
                            rows_v, sem).wait()
      pltpu.sync_copy(rows_v, feat_hbm.at[pl.ds(ebase, PER_W), pl.ds(col0, D)])


def _scatter_body(dst_hbm, eo_hbm, out_hbm, idx_v, rows_v, zbuf_v, acc_sh,
                  sem_ld, sem_add):
  c = lax.axis_index("c")
  s = lax.axis_index("s")
  w = s * NC + c
  row0 = s * ROWS_PER_TILE
  active = w < NACT

  zero = jnp.zeros((D,), jnp.float32)

  def zi(i, carry):
    zbuf_v[i, :] = zero
    return carry

  lax.fori_loop(0, ROWS_PER_TILE, zi, 0)

  @pl.when(active)
  def _():
    pltpu.async_copy(eo_hbm.at[pl.ds(w * PER_W, PER_W)], rows_v, sem_ld)
    pltpu.sync_copy(dst_hbm.at[pl.ds(w * NCHUNK, NCHUNK)], idx_v)

  pltpu.sync_copy(zbuf_v, acc_sh.at[pl.ds(row0, ROWS_PER_TILE)])

  @pl.when(active)
  def _():
    pltpu.make_async_copy(eo_hbm.at[pl.ds(w * PER_W, PER_W)], rows_v,
                          sem_ld).wait()

  plsc.subcore_barrier()

  @pl.when(active)
  def _():
    def fire(j, carry):
      pltpu.async_copy(rows_v.at[pl.ds(j * CHUNK, CHUNK)],
                       acc_sh.at[idx_v.at[j]], sem_add, add=True)
      return carry

    lax.fori_loop(0, NCHUNK, fire, 0)
    pltpu.make_async_copy(eo_hbm.at[pl.ds(w * PER_W, PER_W)], rows_v,
                          sem_add).wait()

  plsc.subcore_barrier()
  pltpu.sync_copy(acc_sh.at[pl.ds(row0, ROWS_PER_TILE)], zbuf_v)
  pltpu.sync_copy(zbuf_v, out_hbm.at[c, pl.ds(row0, ROWS_PER_TILE)])


def _dense_body(emb_ref, feat_ref, nrm_ref, whh_ref, w1b_ref, sw2_ref,
                r_ref, s_ref, out_ref):
  f32 = jnp.float32
  hh = jax.nn.relu(jnp.dot(emb_ref[...], whh_ref[...],
                           preferred_element_type=f32))
  h1 = hh[:, :D]
  h2 = hh[:, D:]
  m1 = jnp.dot(feat_ref[...], w1b_ref[...], preferred_element_type=f32)
  h1r = jnp.dot(h1, r_ref[...], preferred_element_type=f32)
  m2 = jnp.dot(h1r * m1, sw2_ref[...], preferred_element_type=f32)
  h2r = jnp.dot(h2, r_ref[...], preferred_element_type=f32)
  g = jnp.dot(h2r * m2, s_ref[...], preferred_element_type=f32)
  out_ref[...] = (TANH_NORM * jnp.tanh(g)) * nrm_ref[...]


def _combine_body(p_ref, o_ref):
  o_ref[...] = p_ref[0, :N_NODES] + p_ref[1, :N_NODES]


def kernel(x, edge_index, edge_vec, emb, norm, num_nodes,
           fc_w1, fc_w2, fc2_w1, fc2_w2):
  del edge_vec, num_nodes
  f32 = jnp.float32
  dst2 = edge_index[1].reshape(EDGE_ROWS, CHUNK)
  nrm2 = norm.reshape(N_EDGES, 1)

  w1a = fc_w1 * (RELU_NORM / np.sqrt(NUM_BASIS))
  w2a = fc2_w1 * (RELU_NORM / np.sqrt(NUM_BASIS))
  whh = jnp.concatenate([w1a, w2a], axis=1)
  s1 = (1.0 / np.sqrt(fc_w2.shape[0])) * (1.0 / np.sqrt(2 * D))
  w1b = fc_w2.reshape(D, 2 * D, D).transpose(1, 0, 2).reshape(2 * D, D * D) * s1
  s2 = (1.0 / np.sqrt(fc2_w2.shape[0])) * (1.0 / np.sqrt(D))
  w2b = fc2_w2.reshape(D, D, D).transpose(1, 0, 2).reshape(D, D * D) * s2
  r_mat = jnp.repeat(jnp.eye(D, dtype=f32), D, axis=1)
  s_mat = jnp.tile(jnp.eye(D, dtype=f32), (D, 1))
  sw2 = jnp.dot(s_mat, w2b)

  mesh = plsc.VectorSubcoreMesh(core_axis_name="c", subcore_axis_name="s",
                                num_cores=NC, num_subcores=NS)

  gather = pl.kernel(
      _gather_body,
      out_type=jax.ShapeDtypeStruct((N_EDGES, 2 * D), f32),
      mesh=mesh,
      scratch_types=[pltpu.VMEM((PER_W,), jnp.int32),
                     pltpu.VMEM((PER_W, D), f32),
                     pltpu.SemaphoreType.DMA],
      compiler_params=pltpu.CompilerParams(use_tc_tiling_on_sc=False),
  )
  feat = gather(edge_index[0], edge_index[1], x)

  dense = pl.pallas_call(
      _dense_body,
      grid=(GRID,),
      in_specs=[
          pl.BlockSpec((BLK, NUM_BASIS), lambda i: (i, 0)),
          pl.BlockSpec((BLK, 2 * D), lambda i: (i, 0)),
          pl.BlockSpec((BLK, 1), lambda i: (i, 0)),
          pl.BlockSpec((NUM_BASIS, 2 * D), lambda i: (0, 0)),
          pl.BlockSpec((2 * D, D * D), lambda i: (0, 0)),
          pl.BlockSpec((D * D, D * D), lambda i: (0, 0)),
          pl.BlockSpec((D, D * D), lambda i: (0, 0)),
          pl.BlockSpec((D * D, D), lambda i: (0, 0)),
      ],
      out_specs=pl.BlockSpec((BLK, D), lambda i: (i, 0)),
      out_shape=jax.ShapeDtypeStruct((N_EDGES, D), f32),
      compiler_params=pltpu.CompilerParams(
          dimension_semantics=("parallel",)),
  )
  eo = dense(emb, feat, nrm2, whh, w1b, sw2, r_mat, s_mat)

  scatter = pl.kernel(
      _scatter_body,
      out_type=jax.ShapeDtypeStruct((NC, N_PAD, D), f32),
      mesh=mesh,
      scratch_types=[pltpu.VMEM((NCHUNK, CHUNK), jnp.int32),
                     pltpu.VMEM((PER_W, D), f32),
                     pltpu.VMEM((ROWS_PER_TILE, D), f32),
                     pltpu.VMEM_SHARED((N_PAD, D), f32),
                     pltpu.SemaphoreType.DMA,
                     pltpu.SemaphoreType.DMA],
      compiler_params=pltpu.CompilerParams(use_tc_tiling_on_sc=False),
  )
  parts = scatter(dst2, eo)

  combine = pl.pallas_call(
      _combine_body,
      out_shape=jax.ShapeDtypeStruct((N_NODES, D), f32),
  )
  return combine(parts)
